# baseline (speedup 1.0000x reference)
import functools

import jax
import jax.numpy as jnp
from jax import lax
from jax.experimental import pallas as pl
from jax.experimental.pallas import tpu as pltpu
from jax.experimental.pallas import tpu_sc as plsc

NC = 2
NS = 16
NW = NC * NS
L = 16

C = 8
NBUF = 2


def _sc_body(batch, embed, n_chunks, x_hbm, pos_hbm, out_hbm, pos_v, x_v, *sems):
    sin = sems[:NBUF]
    sout = sems[NBUF:]
    wid = lax.axis_index("s") * NC + lax.axis_index("c")
    base = wid * (n_chunks * C)

    def start_in(ci, k):
        s0 = base + ci * C
        pltpu.async_copy(pos_hbm.at[pl.ds(s0, C), :], pos_v.at[k], sin[k])
        for b in range(batch):
            pltpu.async_copy(x_hbm.at[b, pl.ds(s0, C), :], x_v.at[k, b], sin[k])

    def wait_in(k):
        pltpu.make_async_copy(pos_hbm.at[pl.ds(0, C), :], pos_v.at[k], sin[k]).wait()
        for b in range(batch):
            pltpu.make_async_copy(
                x_hbm.at[0, pl.ds(0, C), :], x_v.at[k, b], sin[k]
            ).wait()

    def start_out(ci, k):
        s0 = base + ci * C
        for b in range(batch):
            pltpu.async_copy(x_v.at[k, b], out_hbm.at[b, pl.ds(s0, C), :], sout[k])

    def wait_out(k):
        for b in range(batch):
            pltpu.make_async_copy(
                x_v.at[k, b], out_hbm.at[0, pl.ds(0, C), :], sout[k]
            ).wait()

    def compute(k):
        def row(r, _):
            @plsc.parallel_loop(0, embed, step=L, unroll=8)
            def _(j):
                vp = pos_v[k, r, pl.ds(j, L)]
                for b in range(batch):
                    x_v[k, b, r, pl.ds(j, L)] = x_v[k, b, r, pl.ds(j, L)] + vp

            return 0

        lax.fori_loop(0, C, row, 0)

    for k in range(NBUF):
        start_in(k, k)

    def outer(t, _):
        ci_base = t * NBUF
        for k in range(NBUF):
            ci = ci_base + k
            wait_in(k)
            compute(k)
            start_out(ci, k)

            @pl.when(ci + NBUF < n_chunks)
            def _():
                wait_out(k)
                start_in(ci + NBUF, k)

        return 0

    lax.fori_loop(0, n_chunks // NBUF, outer, 0)
    for k in range(NBUF):
        wait_out(k)


def kernel(x, position_embeddings):
    batch, seq_len, embed = x.shape
    pos = position_embeddings[:seq_len]
    n_chunks = seq_len // (NW * C)
    mesh = plsc.VectorSubcoreMesh(core_axis_name="c", subcore_axis_name="s")
    body = functools.partial(_sc_body, batch, embed, n_chunks)
    return pl.kernel(
        body,
        out_type=jax.ShapeDtypeStruct((batch, seq_len, embed), x.dtype),
        mesh=mesh,
        scratch_types=[
            pltpu.VMEM((NBUF, C, embed), jnp.float32),
            pltpu.VMEM((NBUF, batch, C, embed), jnp.float32),
        ]
        + [pltpu.SemaphoreType.DMA] * (2 * NBUF),
    )(x, pos)


# speedup vs baseline: 1.0158x; 1.0158x over previous
import functools

import jax
import jax.numpy as jnp
from jax import lax
from jax.experimental import pallas as pl
from jax.experimental.pallas import tpu as pltpu
from jax.experimental.pallas import tpu_sc as plsc

NC = 2
NS = 16
NW = NC * NS
L = 16

C = 4
NBUF = 4


def _sc_body(batch, embed, n_chunks, x_hbm, pos_hbm, out_hbm, pos_v, x_v, *sems):
    sin = sems[:NBUF]
    sout = sems[NBUF:]
    wid = lax.axis_index("s") * NC + lax.axis_index("c")
    base = wid * (n_chunks * C)

    def start_in(ci, k):
        s0 = base + ci * C
        pltpu.async_copy(pos_hbm.at[pl.ds(s0, C), :], pos_v.at[k], sin[k])
        for b in range(batch):
            pltpu.async_copy(x_hbm.at[b, pl.ds(s0, C), :], x_v.at[k, b], sin[k])

    def wait_in(k):
        pltpu.make_async_copy(pos_hbm.at[pl.ds(0, C), :], pos_v.at[k], sin[k]).wait()
        for b in range(batch):
            pltpu.make_async_copy(
                x_hbm.at[0, pl.ds(0, C), :], x_v.at[k, b], sin[k]
            ).wait()

    def start_out(ci, k):
        s0 = base + ci * C
        for b in range(batch):
            pltpu.async_copy(x_v.at[k, b], out_hbm.at[b, pl.ds(s0, C), :], sout[k])

    def wait_out(k):
        for b in range(batch):
            pltpu.make_async_copy(
                x_v.at[k, b], out_hbm.at[0, pl.ds(0, C), :], sout[k]
            ).wait()

    def compute(k):
        def row(r, _):
            @plsc.parallel_loop(0, embed, step=L, unroll=8)
            def _(j):
                vp = pos_v[k, r, pl.ds(j, L)]
                for b in range(batch):
                    plsc.addupdate(x_v.at[k, b, r, pl.ds(j, L)], vp)

            return 0

        lax.fori_loop(0, C, row, 0)

    for k in range(NBUF):
        start_in(k, k)

    def outer(t, _):
        ci_base = t * NBUF
        for k in range(NBUF):
            ci = ci_base + k
            wait_in(k)
            compute(k)
            start_out(ci, k)

            @pl.when(ci + NBUF < n_chunks)
            def _():
                wait_out(k)
                start_in(ci + NBUF, k)

        return 0

    lax.fori_loop(0, n_chunks // NBUF, outer, 0)
    for k in range(NBUF):
        wait_out(k)


def kernel(x, position_embeddings):
    batch, seq_len, embed = x.shape
    pos = position_embeddings[:seq_len]
    n_chunks = seq_len // (NW * C)
    mesh = plsc.VectorSubcoreMesh(core_axis_name="c", subcore_axis_name="s")
    body = functools.partial(_sc_body, batch, embed, n_chunks)
    return pl.kernel(
        body,
        out_type=jax.ShapeDtypeStruct((batch, seq_len, embed), x.dtype),
        mesh=mesh,
        scratch_types=[
            pltpu.VMEM((NBUF, C, embed), jnp.float32),
            pltpu.VMEM((NBUF, batch, C, embed), jnp.float32),
        ]
        + [pltpu.SemaphoreType.DMA] * (2 * NBUF),
    )(x, pos)


# speedup vs baseline: 1.0186x; 1.0028x over previous
import functools

import jax
import jax.numpy as jnp
from jax import lax
from jax.experimental import pallas as pl
from jax.experimental.pallas import tpu as pltpu
from jax.experimental.pallas import tpu_sc as plsc

NC = 2
NS = 16
NW = NC * NS
L = 16

C = 2
NBUF = 8


def _sc_body(batch, embed, n_chunks, x_hbm, pos_hbm, out_hbm, pos_v, x_v, *sems):
    sin = sems[:NBUF]
    sout = sems[NBUF:]
    wid = lax.axis_index("s") * NC + lax.axis_index("c")
    base = wid * (n_chunks * C)

    def start_in(ci, k):
        s0 = base + ci * C
        pltpu.async_copy(pos_hbm.at[pl.ds(s0, C), :], pos_v.at[k], sin[k])
        for b in range(batch):
            pltpu.async_copy(x_hbm.at[b, pl.ds(s0, C), :], x_v.at[k, b], sin[k])

    def wait_in(k):
        pltpu.make_async_copy(pos_hbm.at[pl.ds(0, C), :], pos_v.at[k], sin[k]).wait()
        for b in range(batch):
            pltpu.make_async_copy(
                x_hbm.at[0, pl.ds(0, C), :], x_v.at[k, b], sin[k]
            ).wait()

    def start_out(ci, k):
        s0 = base + ci * C
        for b in range(batch):
            pltpu.async_copy(x_v.at[k, b], out_hbm.at[b, pl.ds(s0, C), :], sout[k])

    def wait_out(k):
        for b in range(batch):
            pltpu.make_async_copy(
                x_v.at[k, b], out_hbm.at[0, pl.ds(0, C), :], sout[k]
            ).wait()

    def compute(k):
        def row(r, _):
            @plsc.parallel_loop(0, embed, step=L, unroll=8)
            def _(j):
                vp = pos_v[k, r, pl.ds(j, L)]
                for b in range(batch):
                    plsc.addupdate(x_v.at[k, b, r, pl.ds(j, L)], vp)

            return 0

        lax.fori_loop(0, C, row, 0)

    for k in range(NBUF):
        start_in(k, k)

    def outer(t, _):
        ci_base = t * NBUF
        for k in range(NBUF):
            ci = ci_base + k
            wait_in(k)
            compute(k)
            start_out(ci, k)

            @pl.when(ci + NBUF < n_chunks)
            def _():
                wait_out(k)
                start_in(ci + NBUF, k)

        return 0

    lax.fori_loop(0, n_chunks // NBUF, outer, 0)
    for k in range(NBUF):
        wait_out(k)


def kernel(x, position_embeddings):
    batch, seq_len, embed = x.shape
    pos = position_embeddings[:seq_len]
    n_chunks = seq_len // (NW * C)
    mesh = plsc.VectorSubcoreMesh(core_axis_name="c", subcore_axis_name="s")
    body = functools.partial(_sc_body, batch, embed, n_chunks)
    return pl.kernel(
        body,
        out_type=jax.ShapeDtypeStruct((batch, seq_len, embed), x.dtype),
        mesh=mesh,
        scratch_types=[
            pltpu.VMEM((NBUF, C, embed), jnp.float32),
            pltpu.VMEM((NBUF, batch, C, embed), jnp.float32),
        ]
        + [pltpu.SemaphoreType.DMA] * (2 * NBUF),
    )(x, pos)
